# baseline (speedup 1.0000x reference)
import functools

import jax
import jax.numpy as jnp
from jax import lax
from jax.experimental import pallas as pl
from jax.experimental.pallas import tpu as pltpu
from jax.experimental.pallas import tpu_sc as plsc

N = 10000
DEG = 32
D = 128
S = 8
EPS = 1e-06

NC, NS, L = 2, 16, 16
NW = NC * NS
R_BIG, R_SML = 320, 304
NBIG = 17
QUADS_SML = R_SML // 4
GWIN = R_BIG // 4 + 8
GROWS = 2560

_KS0 = 0
_KS1 = 42
_KS2 = 0x1BD11BDA ^ _KS0 ^ _KS1
_ROTS = ((13, 15, 26, 6), (17, 29, 16, 24))


def _shr(x, n):
    return lax.shift_right_logical(x, jnp.full(x.shape, n, jnp.int32))


def _rotl(x, n):
    return jnp.left_shift(x, n) | _shr(x, 32 - n)


def _threefry_bits(cnt):
    ks = (jnp.int32(_KS0), jnp.int32(_KS1), jnp.int32(_KS2))
    x0 = jnp.zeros_like(cnt) + ks[0]
    x1 = cnt + ks[1]
    for rnd in range(5):
        for r in _ROTS[rnd % 2]:
            x0 = x0 + x1
            x1 = _rotl(x1, r) ^ x0
        x0 = x0 + ks[(rnd + 1) % 3]
        x1 = x1 + ks[(rnd + 2) % 3] + jnp.int32(rnd + 1)
    return x0 ^ x1


def _col_body(e_ref, o_ref):
    o_ref[...] = e_ref[1]


def _col_extract(edge_index):
    return pl.pallas_call(
        _col_body,
        out_shape=jax.ShapeDtypeStruct((N * DEG,), jnp.int32),
    )(edge_index)


LWBLK = 1024
NPAD = 10240


def _logw_body(x_ref, o_ref):
    xb = x_ref[...]
    lw = jnp.log(jnp.sum(xb * xb, axis=1, keepdims=True) + EPS)
    o_ref[...] = jnp.reshape(lw, (LWBLK,))


def _log_weights(x):
    return pl.pallas_call(
        _logw_body,
        out_shape=jax.ShapeDtypeStruct((NPAD,), jnp.float32),
        grid=(NPAD // LWBLK,),
        in_specs=[pl.BlockSpec((LWBLK, D), lambda g: (g, 0))],
        out_specs=pl.BlockSpec((LWBLK,), lambda g: (g,)),
        compiler_params=pltpu.CompilerParams(
            dimension_semantics=("arbitrary",)),
    )(x)


def _gum_body(o_ref):
    g = pl.program_id(0)
    blk = GROWS // 4
    r = lax.broadcasted_iota(jnp.int32, (blk, D), 0)
    c = lax.broadcasted_iota(jnp.int32, (blk, D), 1)
    cnt = (g * blk + r) * D + c
    bits = _threefry_bits(cnt)
    fl = _shr(bits, 9) | jnp.full(bits.shape, 0x3F800000, jnp.int32)
    uf = lax.bitcast_convert_type(fl, jnp.float32) - jnp.float32(1.0)
    mn = jnp.float32(1e-20)
    u = jnp.maximum(mn, uf * (jnp.float32(1.0) - mn) + mn)
    o_ref[...] = -jnp.log(-jnp.log(u))


def _gumbel_table():
    return pl.pallas_call(
        _gum_body,
        out_shape=jax.ShapeDtypeStruct((GROWS, D), jnp.float32),
        grid=(4,),
        out_specs=pl.BlockSpec((GROWS // 4, D), lambda g: (g, 0)),
    )()


def _sc_body(logw_hbm, col_hbm, gum_hbm, dst_hbm, src_hbm,
             logw_v, col_v, gum_v, dst_v, src_v):
    w = lax.axis_index("s") * NC + lax.axis_index("c")
    big = w < NBIG
    base = R_SML * w + (R_BIG - R_SML) * jnp.minimum(w, NBIG)
    lanes = lax.iota(jnp.int32, L)
    m8 = lanes < S
    gstart = pl.multiple_of((base // 32) * 8, 8)
    gdelta = base // 4 - gstart

    pltpu.sync_copy(logw_hbm, logw_v)
    pltpu.sync_copy(gum_hbm.at[pl.ds(gstart, GWIN)], gum_v)

    @pl.when(big)
    def _():
        pltpu.sync_copy(col_hbm.at[pl.ds(base * DEG, R_BIG * DEG)],
                        col_v.at[pl.ds(0, R_BIG * DEG)])

    @pl.when(jnp.logical_not(big))
    def _():
        pltpu.sync_copy(col_hbm.at[pl.ds(base * DEG, R_SML * DEG)],
                        col_v.at[pl.ds(0, R_SML * DEG)])

    def do_row(r):
        off = r * DEG
        grow = gdelta + off // 128
        gcol = off % 128
        iA = col_v[pl.ds(off, L)]
        iB = col_v[pl.ds(off + L, L)]
        gA = gum_v[grow, pl.ds(gcol, L)]
        gB = gum_v[grow, pl.ds(gcol + L, L)]
        kA = plsc.load_gather(logw_v, [iA]) + gA
        kB = plsc.load_gather(logw_v, [iB]) + gB
        sA, vA = plsc.sort_key_val(kA, iA, descending=True)
        sB, vB = plsc.sort_key_val(kB, iB)
        take = sA >= sB
        kM = jnp.where(take, sA, sB)
        vM = jnp.where(take, vA, vB)
        _, top = plsc.sort_key_val(kM, vM, descending=True)
        o = r * S + lanes
        plsc.store_scatter(dst_v, [o], top, mask=m8)
        plsc.store_scatter(src_v, [o], jnp.zeros((L,), jnp.int32) + (base + r),
                           mask=m8)

    @plsc.parallel_loop(0, R_SML, 1, unroll=16)
    def _(r):
        do_row(r)

    @pl.when(big)
    def _():
        @plsc.parallel_loop(R_SML, R_BIG, 1, unroll=16)
        def _(r):
            do_row(r)
        pltpu.sync_copy(dst_v.at[pl.ds(0, R_BIG * S)],
                        dst_hbm.at[pl.ds(base * S, R_BIG * S)])
        pltpu.sync_copy(src_v.at[pl.ds(0, R_BIG * S)],
                        src_hbm.at[pl.ds(base * S, R_BIG * S)])

    @pl.when(jnp.logical_not(big))
    def _():
        pltpu.sync_copy(dst_v.at[pl.ds(0, R_SML * S)],
                        dst_hbm.at[pl.ds(base * S, R_SML * S)])
        pltpu.sync_copy(src_v.at[pl.ds(0, R_SML * S)],
                        src_hbm.at[pl.ds(base * S, R_SML * S)])


def _sc_sample(logw, col, gum):
    mesh = plsc.VectorSubcoreMesh(core_axis_name="c", subcore_axis_name="s")
    k = functools.partial(
        pl.kernel,
        out_type=(
            jax.ShapeDtypeStruct((N * S,), jnp.int32),
            jax.ShapeDtypeStruct((N * S,), jnp.int32),
        ),
        mesh=mesh,
        compiler_params=pltpu.CompilerParams(needs_layout_passes=False),
        scratch_types=[
            pltpu.VMEM((NPAD,), jnp.float32),
            pltpu.VMEM((R_BIG * DEG,), jnp.int32),
            pltpu.VMEM((GWIN, D), jnp.float32),
            pltpu.VMEM((R_BIG * S,), jnp.int32),
            pltpu.VMEM((R_BIG * S,), jnp.int32),
        ],
    )(_sc_body)
    return k(logw, col, gum)


def kernel(x, edge_index):
    col = _col_extract(edge_index)
    logw = _log_weights(x)
    gum = _gumbel_table()
    dst, src = _sc_sample(logw, col, gum)
    return jnp.stack([src, dst])


# speedup vs baseline: 1.1396x; 1.1396x over previous
import functools

import jax
import jax.numpy as jnp
from jax import lax
from jax.experimental import pallas as pl
from jax.experimental.pallas import tpu as pltpu
from jax.experimental.pallas import tpu_sc as plsc

N = 10000
DEG = 32
D = 128
S = 8
EPS = 1e-06

NC, NS, L = 2, 16, 16
NW = NC * NS
R_BIG, R_SML = 320, 304
NBIG = 17
QUADS_SML = R_SML // 4
GWIN = R_BIG // 4 + 8
GROWS = 2560

_KS0 = 0
_KS1 = 42
_KS2 = 0x1BD11BDA ^ _KS0 ^ _KS1
_ROTS = ((13, 15, 26, 6), (17, 29, 16, 24))


def _shr(x, n):
    return lax.shift_right_logical(x, jnp.full(x.shape, n, jnp.int32))


def _rotl(x, n):
    return jnp.left_shift(x, n) | _shr(x, 32 - n)


def _threefry_bits(cnt):
    ks = (jnp.int32(_KS0), jnp.int32(_KS1), jnp.int32(_KS2))
    x0 = jnp.zeros_like(cnt) + ks[0]
    x1 = cnt + ks[1]
    for rnd in range(5):
        for r in _ROTS[rnd % 2]:
            x0 = x0 + x1
            x1 = _rotl(x1, r) ^ x0
        x0 = x0 + ks[(rnd + 1) % 3]
        x1 = x1 + ks[(rnd + 2) % 3] + jnp.int32(rnd + 1)
    return x0 ^ x1


def _col_body(e_ref, o_ref):
    o_ref[...] = e_ref[1]


def _col_extract(edge_index):
    return pl.pallas_call(
        _col_body,
        out_shape=jax.ShapeDtypeStruct((N * DEG,), jnp.int32),
    )(edge_index)


NPAD = 10240
TCG = 10
LWBLK = NPAD // TCG
GBLK = GROWS // TCG


def _tc_body(x_ref, lw_ref, gum_ref):
    g = pl.program_id(0)
    r = lax.broadcasted_iota(jnp.int32, (GBLK, D), 0)
    c = lax.broadcasted_iota(jnp.int32, (GBLK, D), 1)
    cnt = (g * GBLK + r) * D + c
    bits = _threefry_bits(cnt)
    fl = _shr(bits, 9) | jnp.full(bits.shape, 0x3F800000, jnp.int32)
    uf = lax.bitcast_convert_type(fl, jnp.float32) - jnp.float32(1.0)
    mn = jnp.float32(1e-20)
    u = jnp.maximum(mn, uf * (jnp.float32(1.0) - mn) + mn)
    gum_ref[...] = -jnp.log(-jnp.log(u))
    xb = x_ref[...]
    lw = jnp.log(jnp.sum(xb * xb, axis=1, keepdims=True) + EPS)
    lw_ref[...] = jnp.reshape(lw, (LWBLK,))


def _tc_tables(x):
    return pl.pallas_call(
        _tc_body,
        out_shape=(
            jax.ShapeDtypeStruct((NPAD,), jnp.float32),
            jax.ShapeDtypeStruct((GROWS, D), jnp.float32),
        ),
        grid=(TCG,),
        in_specs=[pl.BlockSpec((LWBLK, D), lambda g: (g, 0))],
        out_specs=(
            pl.BlockSpec((LWBLK,), lambda g: (g,)),
            pl.BlockSpec((GBLK, D), lambda g: (g, 0)),
        ),
    )(x)


def _sc_body(logw_hbm, col_hbm, gum_hbm, dst_hbm, src_hbm,
             logw_v, col_v, gum_v, dst_v, src_v):
    w = lax.axis_index("s") * NC + lax.axis_index("c")
    big = w < NBIG
    base = R_SML * w + (R_BIG - R_SML) * jnp.minimum(w, NBIG)
    lanes = lax.iota(jnp.int32, L)
    m8 = lanes < S
    gstart = pl.multiple_of((base // 32) * 8, 8)
    gdelta = base // 4 - gstart

    pltpu.sync_copy(logw_hbm, logw_v)
    pltpu.sync_copy(gum_hbm.at[pl.ds(gstart, GWIN)], gum_v)

    @pl.when(big)
    def _():
        pltpu.sync_copy(col_hbm.at[pl.ds(base * DEG, R_BIG * DEG)],
                        col_v.at[pl.ds(0, R_BIG * DEG)])

    @pl.when(jnp.logical_not(big))
    def _():
        pltpu.sync_copy(col_hbm.at[pl.ds(base * DEG, R_SML * DEG)],
                        col_v.at[pl.ds(0, R_SML * DEG)])

    def do_row(r):
        off = r * DEG
        grow = gdelta + off // 128
        gcol = off % 128
        iA = col_v[pl.ds(off, L)]
        iB = col_v[pl.ds(off + L, L)]
        gA = gum_v[grow, pl.ds(gcol, L)]
        gB = gum_v[grow, pl.ds(gcol + L, L)]
        kA = plsc.load_gather(logw_v, [iA]) + gA
        kB = plsc.load_gather(logw_v, [iB]) + gB
        sA, vA = plsc.sort_key_val(kA, iA, descending=True)
        sB, vB = plsc.sort_key_val(kB, iB)
        take = sA >= sB
        kM = jnp.where(take, sA, sB)
        vM = jnp.where(take, vA, vB)
        _, top = plsc.sort_key_val(kM, vM, descending=True)
        o = r * S + lanes
        plsc.store_scatter(dst_v, [o], top, mask=m8)
        plsc.store_scatter(src_v, [o], jnp.zeros((L,), jnp.int32) + (base + r),
                           mask=m8)

    @plsc.parallel_loop(0, R_SML, 1, unroll=8)
    def _(r):
        do_row(r)

    @pl.when(big)
    def _():
        @plsc.parallel_loop(R_SML, R_BIG, 1, unroll=8)
        def _(r):
            do_row(r)
        pltpu.sync_copy(dst_v.at[pl.ds(0, R_BIG * S)],
                        dst_hbm.at[pl.ds(base * S, R_BIG * S)])
        pltpu.sync_copy(src_v.at[pl.ds(0, R_BIG * S)],
                        src_hbm.at[pl.ds(base * S, R_BIG * S)])

    @pl.when(jnp.logical_not(big))
    def _():
        pltpu.sync_copy(dst_v.at[pl.ds(0, R_SML * S)],
                        dst_hbm.at[pl.ds(base * S, R_SML * S)])
        pltpu.sync_copy(src_v.at[pl.ds(0, R_SML * S)],
                        src_hbm.at[pl.ds(base * S, R_SML * S)])


def _sc_sample(logw, col, gum):
    mesh = plsc.VectorSubcoreMesh(core_axis_name="c", subcore_axis_name="s")
    k = functools.partial(
        pl.kernel,
        out_type=(
            jax.ShapeDtypeStruct((N * S,), jnp.int32),
            jax.ShapeDtypeStruct((N * S,), jnp.int32),
        ),
        mesh=mesh,
        compiler_params=pltpu.CompilerParams(needs_layout_passes=False),
        scratch_types=[
            pltpu.VMEM((NPAD,), jnp.float32),
            pltpu.VMEM((R_BIG * DEG,), jnp.int32),
            pltpu.VMEM((GWIN, D), jnp.float32),
            pltpu.VMEM((R_BIG * S,), jnp.int32),
            pltpu.VMEM((R_BIG * S,), jnp.int32),
        ],
    )(_sc_body)
    return k(logw, col, gum)


def _pack_body(s_ref, d_ref, o_ref):
    o_ref[0, :] = s_ref[...]
    o_ref[1, :] = d_ref[...]


def _pack(src, dst):
    return pl.pallas_call(
        _pack_body,
        out_shape=jax.ShapeDtypeStruct((2, N * S), jnp.int32),
    )(src, dst)


def kernel(x, edge_index):
    col = _col_extract(edge_index)
    logw, gum = _tc_tables(x)
    dst, src = _sc_sample(logw, col, gum)
    return _pack(src, dst)


# speedup vs baseline: 1.2136x; 1.0649x over previous
import functools

import jax
import jax.numpy as jnp
from jax import lax
from jax.experimental import pallas as pl
from jax.experimental.pallas import tpu as pltpu
from jax.experimental.pallas import tpu_sc as plsc

N = 10000
DEG = 32
D = 128
S = 8
EPS = 1e-06

NC, NS, L = 2, 16, 16
NW = NC * NS
R_BIG, R_SML = 320, 304
NBIG = 17
QUADS_SML = R_SML // 4
GWIN = R_BIG // 4 + 8
GROWS = 2560

_KS0 = 0
_KS1 = 42
_KS2 = 0x1BD11BDA ^ _KS0 ^ _KS1
_ROTS = ((13, 15, 26, 6), (17, 29, 16, 24))


def _shr(x, n):
    return lax.shift_right_logical(x, jnp.full(x.shape, n, jnp.int32))


def _rotl(x, n):
    return jnp.left_shift(x, n) | _shr(x, 32 - n)


def _threefry_bits(cnt):
    ks = (jnp.int32(_KS0), jnp.int32(_KS1), jnp.int32(_KS2))
    x0 = jnp.zeros_like(cnt) + ks[0]
    x1 = cnt + ks[1]
    for rnd in range(5):
        for r in _ROTS[rnd % 2]:
            x0 = x0 + x1
            x1 = _rotl(x1, r) ^ x0
        x0 = x0 + ks[(rnd + 1) % 3]
        x1 = x1 + ks[(rnd + 2) % 3] + jnp.int32(rnd + 1)
    return x0 ^ x1


def _col_body(e_ref, o_ref):
    o_ref[...] = e_ref[1]


def _col_extract(edge_index):
    return pl.pallas_call(
        _col_body,
        out_shape=jax.ShapeDtypeStruct((N * DEG,), jnp.int32),
    )(edge_index)


NPAD = 10240
TCG = 10
LWBLK = NPAD // TCG
GBLK = GROWS // TCG


def _tc_body(x_ref, lw_ref, gum_ref):
    g = pl.program_id(0)
    r = lax.broadcasted_iota(jnp.int32, (GBLK, D), 0)
    c = lax.broadcasted_iota(jnp.int32, (GBLK, D), 1)
    cnt = (g * GBLK + r) * D + c
    bits = _threefry_bits(cnt)
    fl = _shr(bits, 9) | jnp.full(bits.shape, 0x3F800000, jnp.int32)
    uf = lax.bitcast_convert_type(fl, jnp.float32) - jnp.float32(1.0)
    mn = jnp.float32(1e-20)
    u = jnp.maximum(mn, uf * (jnp.float32(1.0) - mn) + mn)
    gum_ref[...] = -jnp.log(-jnp.log(u))
    xb = x_ref[...]
    lw = jnp.log(jnp.sum(xb * xb, axis=1, keepdims=True) + EPS)
    lw_ref[...] = jnp.reshape(jnp.transpose(lw, (1, 0)), (LWBLK,))


def _tc_tables(x):
    return pl.pallas_call(
        _tc_body,
        out_shape=(
            jax.ShapeDtypeStruct((NPAD,), jnp.float32),
            jax.ShapeDtypeStruct((GROWS, D), jnp.float32),
        ),
        grid=(TCG,),
        in_specs=[pl.BlockSpec((LWBLK, D), lambda g: (g, 0))],
        out_specs=(
            pl.BlockSpec((LWBLK,), lambda g: (g,)),
            pl.BlockSpec((GBLK, D), lambda g: (g, 0)),
        ),
    )(x)


def _sc_body(logw_hbm, col_hbm, gum_hbm, dst_hbm, src_hbm,
             logw_v, col_v, gum_v, dst_v, src_v):
    w = lax.axis_index("s") * NC + lax.axis_index("c")
    big = w < NBIG
    base = R_SML * w + (R_BIG - R_SML) * jnp.minimum(w, NBIG)
    lanes = lax.iota(jnp.int32, L)
    m8 = lanes < S
    gstart = pl.multiple_of((base // 32) * 8, 8)
    gdelta = base // 4 - gstart

    pltpu.sync_copy(logw_hbm, logw_v)
    pltpu.sync_copy(gum_hbm.at[pl.ds(gstart, GWIN)], gum_v)

    @pl.when(big)
    def _():
        pltpu.sync_copy(col_hbm.at[pl.ds(base * DEG, R_BIG * DEG)],
                        col_v.at[pl.ds(0, R_BIG * DEG)])

    @pl.when(jnp.logical_not(big))
    def _():
        pltpu.sync_copy(col_hbm.at[pl.ds(base * DEG, R_SML * DEG)],
                        col_v.at[pl.ds(0, R_SML * DEG)])

    def do_row(r):
        off = r * DEG
        grow = gdelta + off // 128
        gcol = off % 128
        iA = col_v[pl.ds(off, L)]
        iB = col_v[pl.ds(off + L, L)]
        gA = gum_v[grow, pl.ds(gcol, L)]
        gB = gum_v[grow, pl.ds(gcol + L, L)]
        kA = plsc.load_gather(logw_v, [iA]) + gA
        kB = plsc.load_gather(logw_v, [iB]) + gB
        sA, vA = plsc.sort_key_val(kA, iA, descending=True)
        sB, vB = plsc.sort_key_val(kB, iB)
        take = sA >= sB
        kM = jnp.where(take, sA, sB)
        vM = jnp.where(take, vA, vB)
        _, top = plsc.sort_key_val(kM, vM, descending=True)
        o = r * S + lanes
        plsc.store_scatter(dst_v, [o], top, mask=m8)
        plsc.store_scatter(src_v, [o], jnp.zeros((L,), jnp.int32) + (base + r),
                           mask=m8)

    @plsc.parallel_loop(0, R_SML, 1, unroll=8)
    def _(r):
        do_row(r)

    @pl.when(big)
    def _():
        @plsc.parallel_loop(R_SML, R_BIG, 1, unroll=8)
        def _(r):
            do_row(r)
        pltpu.sync_copy(dst_v.at[pl.ds(0, R_BIG * S)],
                        dst_hbm.at[pl.ds(base * S, R_BIG * S)])
        pltpu.sync_copy(src_v.at[pl.ds(0, R_BIG * S)],
                        src_hbm.at[pl.ds(base * S, R_BIG * S)])

    @pl.when(jnp.logical_not(big))
    def _():
        pltpu.sync_copy(dst_v.at[pl.ds(0, R_SML * S)],
                        dst_hbm.at[pl.ds(base * S, R_SML * S)])
        pltpu.sync_copy(src_v.at[pl.ds(0, R_SML * S)],
                        src_hbm.at[pl.ds(base * S, R_SML * S)])


def _sc_sample(logw, col, gum):
    mesh = plsc.VectorSubcoreMesh(core_axis_name="c", subcore_axis_name="s")
    k = functools.partial(
        pl.kernel,
        out_type=(
            jax.ShapeDtypeStruct((N * S,), jnp.int32),
            jax.ShapeDtypeStruct((N * S,), jnp.int32),
        ),
        mesh=mesh,
        compiler_params=pltpu.CompilerParams(needs_layout_passes=False),
        scratch_types=[
            pltpu.VMEM((NPAD,), jnp.float32),
            pltpu.VMEM((R_BIG * DEG,), jnp.int32),
            pltpu.VMEM((GWIN, D), jnp.float32),
            pltpu.VMEM((R_BIG * S,), jnp.int32),
            pltpu.VMEM((R_BIG * S,), jnp.int32),
        ],
    )(_sc_body)
    return k(logw, col, gum)


def _pack_body(s_ref, d_ref, o_ref):
    o_ref[0, :] = s_ref[...]
    o_ref[1, :] = d_ref[...]


def _pack(src, dst):
    return pl.pallas_call(
        _pack_body,
        out_shape=jax.ShapeDtypeStruct((2, N * S), jnp.int32),
    )(src, dst)


def kernel(x, edge_index):
    col = _col_extract(edge_index)
    logw, gum = _tc_tables(x)
    dst, src = _sc_sample(logw, col, gum)
    return _pack(src, dst)
